# baseline (device time: 48108 ns/iter reference)
import jax
import jax.numpy as jnp
from jax import lax
from jax.experimental import pallas as pl
from jax.experimental.pallas import tpu as pltpu

N_DEV = 8
B = 2
SQ = 256
HQ = 8
DH = 64
BH = B * HQ
SCALE = 0.125
PACK = 128
MLROW = BH // 2


def kernel(x, Wq, Wo, K_ext, V_ext):
    def body(x_ref, wq_ref, wo_ref, k_ref, v_ref, out_ref,
             comm_ref, send_sems, recv_sems):
        my_pos = lax.axis_index("i")

        barrier_sem = pltpu.get_barrier_semaphore()
        for d in (1, 2, 4):
            pl.semaphore_signal(
                barrier_sem, inc=1,
                device_id=(jnp.bitwise_xor(my_pos, d),),
                device_id_type=pl.DeviceIdType.MESH,
            )
        pl.semaphore_wait(barrier_sem, 3)

        os, ls = [], []
        for b in range(B):
            q_b = jnp.dot(x_ref[b].astype(jnp.bfloat16),
                          wq_ref[...].astype(jnp.bfloat16),
                          preferred_element_type=jnp.float32)
            q_b = q_b.astype(jnp.bfloat16)
            for h in range(HQ):
                q_bh = q_b[:, h * DH:(h + 1) * DH]
                k_bh = k_ref[b, :, h, :].astype(jnp.bfloat16)
                v_bh = v_ref[b, :, h, :].astype(jnp.bfloat16)
                s = lax.dot_general(
                    q_bh, k_bh, (((1,), (1,)), ((), ())),
                    preferred_element_type=jnp.float32,
                ) * SCALE
                p = jnp.exp(s)
                os.append(jnp.dot(p.astype(jnp.bfloat16), v_bh,
                                  preferred_element_type=jnp.float32))
                ls.append(jnp.sum(p, axis=1, keepdims=True))

        for j in range(MLROW):
            comm_ref[0, j] = jnp.concatenate(
                [os[2 * j], os[2 * j + 1]], axis=1).astype(jnp.bfloat16)
        l_all = jnp.concatenate(ls, axis=1).astype(jnp.bfloat16)
        comm_ref[0, MLROW] = jnp.concatenate(
            [l_all, jnp.zeros((SQ, PACK - BH), jnp.bfloat16)], axis=1)

        for step, d in enumerate((1, 2, 4)):
            rdma = pltpu.make_async_remote_copy(
                src_ref=comm_ref.at[0],
                dst_ref=comm_ref.at[1 + step],
                send_sem=send_sems.at[step],
                recv_sem=recv_sems.at[step],
                device_id=(jnp.bitwise_xor(my_pos, d),),
                device_id_type=pl.DeviceIdType.MESH,
            )
            rdma.start()
            rdma.wait()
            comm_ref[0] = comm_ref[0] + comm_ref[1 + step]

        for b in range(B):
            cols = []
            for h in range(HQ):
                idx = b * HQ + h
                off = (idx % 2) * DH
                o = comm_ref[0, idx // 2, :, off:off + DH].astype(jnp.float32)
                l = comm_ref[0, MLROW, :, idx:idx + 1].astype(jnp.float32)
                cols.append((o / l).astype(jnp.bfloat16))
            attn_b = jnp.concatenate(cols, axis=1)
            out_ref[b] = jnp.dot(attn_b, wo_ref[...].astype(jnp.bfloat16),
                                 preferred_element_type=jnp.float32)

    return pl.pallas_call(
        body,
        out_shape=jax.ShapeDtypeStruct((B, SQ, 768), jnp.float32),
        in_specs=[
            pl.BlockSpec(memory_space=pltpu.VMEM),
            pl.BlockSpec(memory_space=pltpu.VMEM),
            pl.BlockSpec(memory_space=pltpu.VMEM),
            pl.BlockSpec(memory_space=pltpu.VMEM),
            pl.BlockSpec(memory_space=pltpu.VMEM),
        ],
        out_specs=pl.BlockSpec(memory_space=pltpu.VMEM),
        scratch_shapes=[
            pltpu.VMEM((4, MLROW + 1, SQ, PACK), jnp.bfloat16),
            pltpu.SemaphoreType.DMA((3,)),
            pltpu.SemaphoreType.DMA((3,)),
        ],
        compiler_params=pltpu.CompilerParams(collective_id=0),
    )(x, Wq, Wo, K_ext, V_ext)


# device time: 41880 ns/iter; 1.1487x vs baseline; 1.1487x over previous
import jax
import jax.numpy as jnp
from jax import lax
from jax.experimental import pallas as pl
from jax.experimental.pallas import tpu as pltpu

N_DEV = 8
B = 2
SQ = 256
HQ = 8
DH = 64
BH = B * HQ
SCALE = 0.125
PACK = 128
HROWS = HQ // 2
BROWS = HROWS + 1
NROWS = B * BROWS


def kernel(x, Wq, Wo, K_ext, V_ext):
    def body(x_ref, wq_ref, wo_ref, k_ref, v_ref, out_ref,
             comm_ref, send_sems, recv_sems):
        my_pos = lax.axis_index("i")

        barrier_sem = pltpu.get_barrier_semaphore()
        for d in (1, 2, 4):
            pl.semaphore_signal(
                barrier_sem, inc=1,
                device_id=(jnp.bitwise_xor(my_pos, d),),
                device_id_type=pl.DeviceIdType.MESH,
            )
        pl.semaphore_wait(barrier_sem, 3)

        def attn_partial(b):
            q_b = jnp.dot(x_ref[b], wq_ref[...],
                          preferred_element_type=jnp.float32)
            os, ls = [], []
            for h in range(HQ):
                q_bh = q_b[:, h * DH:(h + 1) * DH]
                k_bh = k_ref[b, :, h, :]
                v_bh = v_ref[b, :, h, :]
                s = lax.dot_general(
                    q_bh, k_bh, (((1,), (1,)), ((), ())),
                    preferred_element_type=jnp.float32,
                ) * SCALE
                p = jnp.exp(s)
                os.append(jnp.dot(p, v_bh,
                                  preferred_element_type=jnp.float32))
                ls.append(jnp.sum(p, axis=1, keepdims=True))
            return os, ls

        def pack(b, os, ls):
            base = b * BROWS
            for j in range(HROWS):
                comm_ref[0, base + j] = jnp.concatenate(
                    [os[2 * j], os[2 * j + 1]], axis=1).astype(jnp.bfloat16)
            l_b = jnp.concatenate(ls, axis=1).astype(jnp.bfloat16)
            comm_ref[0, base + HROWS] = jnp.concatenate(
                [l_b, jnp.zeros((SQ, PACK - HQ), jnp.bfloat16)], axis=1)

        def region_rdma(slot, b, d, sem_idx):
            rows = pl.ds(b * BROWS, BROWS)
            return pltpu.make_async_remote_copy(
                src_ref=comm_ref.at[0, rows],
                dst_ref=comm_ref.at[slot, rows],
                send_sem=send_sems.at[sem_idx],
                recv_sem=recv_sems.at[sem_idx],
                device_id=(jnp.bitwise_xor(my_pos, d),),
                device_id_type=pl.DeviceIdType.MESH,
            )

        os0, ls0 = attn_partial(0)
        pack(0, os0, ls0)
        rdma_a = region_rdma(1, 0, 1, 0)
        rdma_a.start()
        os1, ls1 = attn_partial(1)
        pack(1, os1, ls1)
        rdma_b = region_rdma(1, 1, 1, 1)
        rdma_b.start()
        rdma_a.wait()
        rdma_b.wait()
        comm_ref[0] = comm_ref[0] + comm_ref[1]

        rdma = pltpu.make_async_remote_copy(
            src_ref=comm_ref.at[0],
            dst_ref=comm_ref.at[2],
            send_sem=send_sems.at[2],
            recv_sem=recv_sems.at[2],
            device_id=(jnp.bitwise_xor(my_pos, 2),),
            device_id_type=pl.DeviceIdType.MESH,
        )
        rdma.start()
        rdma.wait()
        comm_ref[0] = comm_ref[0] + comm_ref[2]

        def finalize(b):
            base = b * BROWS
            cols = []
            for h in range(HQ):
                off = (h % 2) * DH
                o = comm_ref[0, base + h // 2, :, off:off + DH].astype(
                    jnp.float32)
                l = comm_ref[0, base + HROWS, :, h:h + 1].astype(jnp.float32)
                cols.append(o / l)
            attn_b = jnp.concatenate(cols, axis=1)
            out_ref[b] = jnp.dot(attn_b, wo_ref[...],
                                 preferred_element_type=jnp.float32)

        rdma_a = region_rdma(3, 0, 4, 3)
        rdma_b = region_rdma(3, 1, 4, 4)
        rdma_a.start()
        rdma_b.start()
        rdma_a.wait()
        comm_ref[0, 0:BROWS] = comm_ref[0, 0:BROWS] + comm_ref[3, 0:BROWS]
        finalize(0)
        rdma_b.wait()
        comm_ref[0, BROWS:NROWS] = (comm_ref[0, BROWS:NROWS]
                                    + comm_ref[3, BROWS:NROWS])
        finalize(1)

    return pl.pallas_call(
        body,
        out_shape=jax.ShapeDtypeStruct((B, SQ, 768), jnp.float32),
        in_specs=[
            pl.BlockSpec(memory_space=pltpu.VMEM),
            pl.BlockSpec(memory_space=pltpu.VMEM),
            pl.BlockSpec(memory_space=pltpu.VMEM),
            pl.BlockSpec(memory_space=pltpu.VMEM),
            pl.BlockSpec(memory_space=pltpu.VMEM),
        ],
        out_specs=pl.BlockSpec(memory_space=pltpu.VMEM),
        scratch_shapes=[
            pltpu.VMEM((4, NROWS, SQ, PACK), jnp.bfloat16),
            pltpu.SemaphoreType.DMA((5,)),
            pltpu.SemaphoreType.DMA((5,)),
        ],
        compiler_params=pltpu.CompilerParams(collective_id=0),
    )(x, Wq, Wo, K_ext, V_ext)


# device time: 40670 ns/iter; 1.1829x vs baseline; 1.0298x over previous
import jax
import jax.numpy as jnp
from jax import lax
from jax.experimental import pallas as pl
from jax.experimental.pallas import tpu as pltpu

N_DEV = 8
B = 2
SQ = 256
HQ = 8
DH = 64
BH = B * HQ
SCALE = 0.125
PACK = 128
HROWS = HQ // 2
LROW = HROWS
NROWS = 2 * HROWS + 1


def kernel(x, Wq, Wo, K_ext, V_ext):
    def body(x_ref, wq_ref, wo_ref, k_ref, v_ref, out_ref,
             comm_ref, send_sems, recv_sems):
        my_pos = lax.axis_index("i")

        barrier_sem = pltpu.get_barrier_semaphore()
        for d in (1, 2, 4):
            pl.semaphore_signal(
                barrier_sem, inc=1,
                device_id=(jnp.bitwise_xor(my_pos, d),),
                device_id_type=pl.DeviceIdType.MESH,
            )
        pl.semaphore_wait(barrier_sem, 3)

        def attn_heads(b, q_b, h0, h1):
            os, ls = [], []
            for h in range(h0, h1):
                q_bh = q_b[:, h * DH:(h + 1) * DH]
                k_bh = k_ref[b, :, h, :]
                v_bh = v_ref[b, :, h, :]
                s = lax.dot_general(
                    q_bh, k_bh, (((1,), (1,)), ((), ())),
                    preferred_element_type=jnp.float32,
                ) * SCALE
                p = jnp.exp(s)
                os.append(jnp.dot(p, v_bh,
                                  preferred_element_type=jnp.float32))
                ls.append(jnp.sum(p, axis=1, keepdims=True))
            return os, ls

        def pack_o(row, os):
            for j in range(len(os) // 2):
                comm_ref[0, row + j] = jnp.concatenate(
                    [os[2 * j], os[2 * j + 1]], axis=1).astype(jnp.bfloat16)

        def region_rdma(slot, row0, nrows, d, sem_idx):
            rows = pl.ds(row0, nrows)
            return pltpu.make_async_remote_copy(
                src_ref=comm_ref.at[0, rows],
                dst_ref=comm_ref.at[slot, rows],
                send_sem=send_sems.at[sem_idx],
                recv_sem=recv_sems.at[sem_idx],
                device_id=(jnp.bitwise_xor(my_pos, d),),
                device_id_type=pl.DeviceIdType.MESH,
            )

        q0 = jnp.dot(x_ref[0], wq_ref[...],
                     preferred_element_type=jnp.float32)
        os_a, ls_a = attn_heads(0, q0, 0, HQ // 2)
        pack_o(0, os_a)
        rdma_a = region_rdma(1, 0, 2, 1, 0)
        rdma_a.start()
        os_b, ls_b = attn_heads(0, q0, HQ // 2, HQ)
        pack_o(2, os_b)
        l0 = jnp.concatenate(ls_a + ls_b, axis=1).astype(jnp.bfloat16)
        comm_ref[0, LROW] = jnp.concatenate(
            [l0, jnp.zeros((SQ, PACK - HQ), jnp.bfloat16)], axis=1)
        rdma_b = region_rdma(1, 2, 2, 1, 1)
        rdma_b.start()
        q1 = jnp.dot(x_ref[1], wq_ref[...],
                     preferred_element_type=jnp.float32)
        os_c, ls_c = attn_heads(1, q1, 0, HQ)
        pack_o(LROW + 1, os_c)
        l1 = jnp.concatenate(ls_c, axis=1).astype(jnp.bfloat16)
        comm_ref[0, LROW, :, HQ:2 * HQ] = l1
        rdma_c = region_rdma(1, LROW, NROWS - LROW, 1, 2)
        rdma_c.start()
        rdma_a.wait()
        rdma_b.wait()
        rdma_c.wait()
        comm_ref[0] = comm_ref[0] + comm_ref[1]

        rdma = region_rdma(2, 0, NROWS, 2, 3)
        rdma.start()
        rdma.wait()
        comm_ref[0] = comm_ref[0] + comm_ref[2]

        def finalize(b):
            o_base = 0 if b == 0 else LROW + 1
            cols = []
            for h in range(HQ):
                off = (h % 2) * DH
                o = comm_ref[0, o_base + h // 2, :, off:off + DH].astype(
                    jnp.float32)
                lane = b * HQ + h
                l = comm_ref[0, LROW, :, lane:lane + 1].astype(jnp.float32)
                cols.append(o / l)
            attn_b = jnp.concatenate(cols, axis=1)
            out_ref[b] = jnp.dot(attn_b, wo_ref[...],
                                 preferred_element_type=jnp.float32)

        rdma_a = region_rdma(3, 0, LROW + 1, 4, 4)
        rdma_b = region_rdma(3, LROW + 1, HROWS, 4, 5)
        rdma_a.start()
        rdma_b.start()
        rdma_a.wait()
        comm_ref[0, 0:LROW + 1] = (comm_ref[0, 0:LROW + 1]
                                   + comm_ref[3, 0:LROW + 1])
        finalize(0)
        rdma_b.wait()
        comm_ref[0, LROW + 1:NROWS] = (comm_ref[0, LROW + 1:NROWS]
                                       + comm_ref[3, LROW + 1:NROWS])
        finalize(1)

    return pl.pallas_call(
        body,
        out_shape=jax.ShapeDtypeStruct((B, SQ, 768), jnp.float32),
        in_specs=[
            pl.BlockSpec(memory_space=pltpu.VMEM),
            pl.BlockSpec(memory_space=pltpu.VMEM),
            pl.BlockSpec(memory_space=pltpu.VMEM),
            pl.BlockSpec(memory_space=pltpu.VMEM),
            pl.BlockSpec(memory_space=pltpu.VMEM),
        ],
        out_specs=pl.BlockSpec(memory_space=pltpu.VMEM),
        scratch_shapes=[
            pltpu.VMEM((4, NROWS, SQ, PACK), jnp.bfloat16),
            pltpu.SemaphoreType.DMA((6,)),
            pltpu.SemaphoreType.DMA((6,)),
        ],
        compiler_params=pltpu.CompilerParams(collective_id=0),
    )(x, Wq, Wo, K_ext, V_ext)


# device time: 31722 ns/iter; 1.5166x vs baseline; 1.2821x over previous
import jax
import jax.numpy as jnp
from jax import lax
from jax.experimental import pallas as pl
from jax.experimental.pallas import tpu as pltpu

N_DEV = 8
B = 2
SQ = 256
HQ = 8
DH = 64
BH = B * HQ
SCALE = 0.125
PACK = 128
REGIONS = ((0, 2), (2, 3), (5, 2), (7, 3))
DSEQ = ((1, 2, 4), (4, 2, 1), (1, 2, 4), (4, 2, 1))
L0ROW = 4
L1ROW = 9
NROWS = 10


def kernel(x, Wq, Wo, K_ext, V_ext):
    def body(x_ref, wq_ref, wo_ref, k_ref, v_ref, out_ref,
             comm_ref, send_sems, recv_sems):
        my_pos = lax.axis_index("i")

        barrier_sem = pltpu.get_barrier_semaphore()
        for d in (1, 2, 4):
            pl.semaphore_signal(
                barrier_sem, inc=1,
                device_id=(jnp.bitwise_xor(my_pos, d),),
                device_id_type=pl.DeviceIdType.MESH,
            )
        pl.semaphore_wait(barrier_sem, 3)

        def attn_heads(b, q_b, h0, h1):
            os, ls = [], []
            for h in range(h0, h1):
                q_bh = q_b[:, h * DH:(h + 1) * DH]
                k_bh = k_ref[b, :, h, :]
                v_bh = v_ref[b, :, h, :]
                s = lax.dot_general(
                    q_bh, k_bh, (((1,), (1,)), ((), ())),
                    preferred_element_type=jnp.float32,
                ) * SCALE
                p = jnp.exp(s)
                os.append(jnp.dot(p, v_bh,
                                  preferred_element_type=jnp.float32))
                ls.append(jnp.sum(p, axis=1, keepdims=True))
            return os, ls

        def pack_o(row, os):
            for j in range(len(os) // 2):
                comm_ref[0, row + j] = jnp.concatenate(
                    [os[2 * j], os[2 * j + 1]], axis=1).astype(jnp.bfloat16)

        def pack_l(row, ls):
            l_col = jnp.concatenate(ls, axis=1).astype(jnp.bfloat16)
            comm_ref[0, row] = jnp.concatenate(
                [l_col, jnp.zeros((SQ, PACK - HQ), jnp.bfloat16)], axis=1)

        def start_step(r, s):
            row0, nrows = REGIONS[r]
            rows = pl.ds(row0, nrows)
            sem = 3 * r + s
            rdma = pltpu.make_async_remote_copy(
                src_ref=comm_ref.at[0, rows],
                dst_ref=comm_ref.at[1 + s, rows],
                send_sem=send_sems.at[sem],
                recv_sem=recv_sems.at[sem],
                device_id=(jnp.bitwise_xor(my_pos, DSEQ[r][s]),),
                device_id_type=pl.DeviceIdType.MESH,
            )
            rdma.start()
            return rdma

        def merge(r, s):
            row0, nrows = REGIONS[r]
            comm_ref[0, row0:row0 + nrows] = (
                comm_ref[0, row0:row0 + nrows]
                + comm_ref[1 + s, row0:row0 + nrows])

        pend = [None] * 4
        q0 = jnp.dot(x_ref[0], wq_ref[...],
                     preferred_element_type=jnp.float32)
        os_a, ls_a = attn_heads(0, q0, 0, HQ // 2)
        pack_o(0, os_a)
        pend[0] = start_step(0, 0)
        os_b, ls_b = attn_heads(0, q0, HQ // 2, HQ)
        pack_o(2, os_b)
        pack_l(L0ROW, ls_a + ls_b)
        pend[1] = start_step(1, 0)
        q1 = jnp.dot(x_ref[1], wq_ref[...],
                     preferred_element_type=jnp.float32)
        os_c, ls_c = attn_heads(1, q1, 0, HQ // 2)
        pack_o(5, os_c)
        pend[2] = start_step(2, 0)
        os_d, ls_d = attn_heads(1, q1, HQ // 2, HQ)
        pack_o(7, os_d)
        pack_l(L1ROW, ls_c + ls_d)
        pend[3] = start_step(3, 0)

        for s in range(2):
            for r in range(4):
                pend[r].wait()
                merge(r, s)
                pend[r] = start_step(r, s + 1)

        def finalize(b):
            o_base, l_row = (0, L0ROW) if b == 0 else (5, L1ROW)
            cols = []
            for h in range(HQ):
                off = (h % 2) * DH
                o = comm_ref[0, o_base + h // 2, :, off:off + DH].astype(
                    jnp.float32)
                l = comm_ref[0, l_row, :, h:h + 1].astype(jnp.float32)
                cols.append(o / l)
            attn_b = jnp.concatenate(cols, axis=1)
            out_ref[b] = jnp.dot(attn_b, wo_ref[...],
                                 preferred_element_type=jnp.float32)

        pend[0].wait()
        merge(0, 2)
        pend[1].wait()
        merge(1, 2)
        finalize(0)
        pend[2].wait()
        merge(2, 2)
        pend[3].wait()
        merge(3, 2)
        finalize(1)

    return pl.pallas_call(
        body,
        out_shape=jax.ShapeDtypeStruct((B, SQ, 768), jnp.float32),
        in_specs=[
            pl.BlockSpec(memory_space=pltpu.VMEM),
            pl.BlockSpec(memory_space=pltpu.VMEM),
            pl.BlockSpec(memory_space=pltpu.VMEM),
            pl.BlockSpec(memory_space=pltpu.VMEM),
            pl.BlockSpec(memory_space=pltpu.VMEM),
        ],
        out_specs=pl.BlockSpec(memory_space=pltpu.VMEM),
        scratch_shapes=[
            pltpu.VMEM((4, NROWS, SQ, PACK), jnp.bfloat16),
            pltpu.SemaphoreType.DMA((12,)),
            pltpu.SemaphoreType.DMA((12,)),
        ],
        compiler_params=pltpu.CompilerParams(collective_id=0),
    )(x, Wq, Wo, K_ext, V_ext)


# device time: 31120 ns/iter; 1.5459x vs baseline; 1.0193x over previous
import jax
import jax.numpy as jnp
from jax import lax
from jax.experimental import pallas as pl
from jax.experimental.pallas import tpu as pltpu

N_DEV = 8
B = 2
SQ = 256
HQ = 8
DH = 64
BH = B * HQ
SCALE = 0.125
PACK = 128
REGIONS = ((0, 2), (2, 2), (4, 2), (6, 2))
DSEQ = (
    (1, 2, 4), (4, 2, 1), (1, 2, 4), (4, 2, 1),
    (2, 1, 4),
)
NROWS = 8


def kernel(x, Wq, Wo, K_ext, V_ext):
    def body(x_ref, wq_ref, wo_ref, k_ref, v_ref, out_ref,
             comm_ref, l_comm_ref, send_sems, recv_sems):
        my_pos = lax.axis_index("i")

        barrier_sem = pltpu.get_barrier_semaphore()
        for d in (1, 2, 4):
            pl.semaphore_signal(
                barrier_sem, inc=1,
                device_id=(jnp.bitwise_xor(my_pos, d),),
                device_id_type=pl.DeviceIdType.MESH,
            )
        pl.semaphore_wait(barrier_sem, 3)

        def attn_heads(b, q_b, h0, h1):
            os, ls = [], []
            for h in range(h0, h1):
                q_bh = q_b[:, h * DH:(h + 1) * DH]
                k_bh = k_ref[b, :, h, :]
                v_bh = v_ref[b, :, h, :]
                s = lax.dot_general(
                    q_bh, k_bh, (((1,), (1,)), ((), ())),
                    preferred_element_type=jnp.float32,
                ) * SCALE
                p = jnp.exp(s)
                os.append(jnp.dot(p, v_bh,
                                  preferred_element_type=jnp.float32))
                ls.append(jnp.sum(p, axis=1, keepdims=True))
            return os, ls

        def pack_o(row, os):
            for j in range(len(os) // 2):
                comm_ref[0, row + j] = jnp.concatenate(
                    [os[2 * j], os[2 * j + 1]], axis=1).astype(jnp.bfloat16)

        def start_step(r, s):
            sem = 3 * r + s
            if r < 4:
                row0, nrows = REGIONS[r]
                rows = pl.ds(row0, nrows)
                src, dst = comm_ref.at[0, rows], comm_ref.at[1 + s, rows]
            else:
                src, dst = l_comm_ref.at[0], l_comm_ref.at[1 + s]
            rdma = pltpu.make_async_remote_copy(
                src_ref=src,
                dst_ref=dst,
                send_sem=send_sems.at[sem],
                recv_sem=recv_sems.at[sem],
                device_id=(jnp.bitwise_xor(my_pos, DSEQ[r][s]),),
                device_id_type=pl.DeviceIdType.MESH,
            )
            rdma.start()
            return rdma

        def merge(r, s):
            if r < 4:
                row0, nrows = REGIONS[r]
                comm_ref[0, row0:row0 + nrows] = (
                    comm_ref[0, row0:row0 + nrows]
                    + comm_ref[1 + s, row0:row0 + nrows])
            else:
                l_comm_ref[0] = l_comm_ref[0] + l_comm_ref[1 + s]

        pend = [None] * 5
        q0 = jnp.dot(x_ref[0], wq_ref[...],
                     preferred_element_type=jnp.float32)
        os_a, ls_a = attn_heads(0, q0, 0, HQ // 2)
        pack_o(0, os_a)
        pend[0] = start_step(0, 0)
        os_b, ls_b = attn_heads(0, q0, HQ // 2, HQ)
        pack_o(2, os_b)
        pend[1] = start_step(1, 0)
        q1 = jnp.dot(x_ref[1], wq_ref[...],
                     preferred_element_type=jnp.float32)
        os_c, ls_c = attn_heads(1, q1, 0, HQ // 2)
        pack_o(4, os_c)
        pend[2] = start_step(2, 0)
        os_d, ls_d = attn_heads(1, q1, HQ // 2, HQ)
        pack_o(6, os_d)
        pend[3] = start_step(3, 0)
        l_comm_ref[0] = jnp.concatenate(
            ls_a + ls_b + ls_c + ls_d, axis=1).astype(jnp.bfloat16)
        pend[4] = start_step(4, 0)

        for s in range(2):
            for r in range(5):
                pend[r].wait()
                merge(r, s)
                pend[r] = start_step(r, s + 1)

        def half_fin(b, half):
            cols = []
            for h in range(4 * half, 4 * half + 4):
                off = (h % 2) * DH
                o = comm_ref[0, 2 * (2 * b + half) + h % 4 // 2, :,
                             off:off + DH].astype(jnp.float32)
                l = l_comm_ref[0][:, b * HQ + h:b * HQ + h + 1].astype(
                    jnp.float32)
                cols.append(o / l)
            attn_half = jnp.concatenate(cols, axis=1)
            return jnp.dot(attn_half, wo_ref[4 * half * DH:
                                             (4 * half + 4) * DH, :],
                           preferred_element_type=jnp.float32)

        pend[4].wait()
        merge(4, 2)
        pend[0].wait()
        merge(0, 2)
        pend[1].wait()
        merge(1, 2)
        out_ref[0] = half_fin(0, 0) + half_fin(0, 1)
        pend[2].wait()
        merge(2, 2)
        part1 = half_fin(1, 0)
        pend[3].wait()
        merge(3, 2)
        out_ref[1] = part1 + half_fin(1, 1)

    return pl.pallas_call(
        body,
        out_shape=jax.ShapeDtypeStruct((B, SQ, 768), jnp.float32),
        in_specs=[
            pl.BlockSpec(memory_space=pltpu.VMEM),
            pl.BlockSpec(memory_space=pltpu.VMEM),
            pl.BlockSpec(memory_space=pltpu.VMEM),
            pl.BlockSpec(memory_space=pltpu.VMEM),
            pl.BlockSpec(memory_space=pltpu.VMEM),
        ],
        out_specs=pl.BlockSpec(memory_space=pltpu.VMEM),
        scratch_shapes=[
            pltpu.VMEM((4, NROWS, SQ, PACK), jnp.bfloat16),
            pltpu.VMEM((4, SQ, BH), jnp.bfloat16),
            pltpu.SemaphoreType.DMA((15,)),
            pltpu.SemaphoreType.DMA((15,)),
        ],
        compiler_params=pltpu.CompilerParams(collective_id=0),
    )(x, Wq, Wo, K_ext, V_ext)
